# baseline (speedup 1.0000x reference)
import functools

import jax
import jax.numpy as jnp
from jax import lax
from jax.experimental import pallas as pl
from jax.experimental.pallas import tpu as pltpu
from jax.experimental.pallas import tpu_sc as plsc

_LANES = 16
_PADW = 4
_TILE = 128
_T = 40


@functools.lru_cache(maxsize=None)
def _build(N: int, E: int, interpret: bool):
    try:
        info = plsc.get_sparse_core_info()
        NC, NS = info.num_cores, info.num_subcores
    except ValueError:
        NC, NS = 2, 16
    NW = NC * NS
    assert NW == 32
    assert E % (_TILE * _T) == 0, E
    NT = E // _TILE
    NCH = NT // _T
    C = _T * _TILE
    M = C // _LANES

    mesh = plsc.VectorSubcoreMesh(
        core_axis_name="c", subcore_axis_name="s", num_cores=NC, num_subcores=NS)

    @functools.partial(
        pl.kernel,
        mesh=mesh,
        out_type=jax.ShapeDtypeStruct((NT, 4, _TILE), jnp.float32),
        scratch_types=[
            pltpu.VMEM((C,), jnp.int32),
            pltpu.VMEM((C,), jnp.int32),
            pltpu.VMEM((C, _PADW), jnp.float32),
            pltpu.VMEM((C, _PADW), jnp.float32),
            pltpu.VMEM((_T, 4, _TILE), jnp.float32),
            pltpu.SemaphoreType.DMA,
            pltpu.SemaphoreType.DMA,
        ],
        compiler_params=pltpu.CompilerParams(
            use_tc_tiling_on_sc=False, needs_layout_passes=False),
        interpret=interpret,
    )
    def k(pos_hbm, ii_hbm, ij_hbm, off_hbm, out_hbm,
          ii_v, ij_v, pos_i_v, pos_j_v, io_v, sem_i, sem_j):
        wid = lax.axis_index("s") * NC + lax.axis_index("c")
        c_lo = lax.shift_right_logical(wid * NCH, 5)
        c_hi = lax.shift_right_logical((wid + 1) * NCH, 5)

        def chunk_body(c, _):
            base = c * C
            t0 = c * _T
            pltpu.sync_copy(ii_hbm.at[pl.ds(base, C)], ii_v)
            pltpu.sync_copy(ij_hbm.at[pl.ds(base, C)], ij_v)
            cp_i = pltpu.async_copy(pos_hbm.at[ii_v], pos_i_v, sem_i)
            cp_j = pltpu.async_copy(pos_hbm.at[ij_v], pos_j_v, sem_j)
            pltpu.sync_copy(off_hbm.at[pl.ds(t0, _T)], io_v)
            cp_i.wait()
            cp_j.wait()

            def group_body(m, _):
                tt = lax.shift_right_logical(m, 3)
                lo = 16 * (m & 7)
                e_vec = m * _LANES + lax.iota(jnp.int32, _LANES)
                for s in range(3):
                    k_vec = jnp.broadcast_to(jnp.int32(s), (_LANES,))
                    pi = plsc.load_gather(pos_i_v, [e_vec, k_vec])
                    pj = plsc.load_gather(pos_j_v, [e_vec, k_vec])
                    io_v[tt, s, pl.ds(lo, _LANES)] = (
                        pj - pi + io_v[tt, s, pl.ds(lo, _LANES)])
                return 0

            lax.fori_loop(0, M, group_body, 0, unroll=False)
            pltpu.sync_copy(io_v, out_hbm.at[pl.ds(t0, _T)])
            return 0

        lax.fori_loop(c_lo, c_hi, chunk_body, 0, unroll=False)

    return k


def kernel(positions, indeces_i, indeces_j, offsets):
    N, _ = positions.shape
    E = indeces_i.shape[0]
    pos_pad = jnp.pad(positions, ((0, 0), (0, _PADW - positions.shape[1])))
    off_t = (jnp.pad(offsets, ((0, 0), (0, 1)))
             .reshape(E // _TILE, _TILE, 4)
             .transpose(0, 2, 1))
    k = _build(N, E, False)
    out_t = k(
        pos_pad,
        indeces_i.astype(jnp.int32),
        indeces_j.astype(jnp.int32),
        off_t,
    )
    return out_t.transpose(0, 2, 1).reshape(E, 4)[:, :3]


# speedup vs baseline: 1.0035x; 1.0035x over previous
import functools

import jax
import jax.numpy as jnp
from jax import lax
from jax.experimental import pallas as pl
from jax.experimental.pallas import tpu as pltpu
from jax.experimental.pallas import tpu_sc as plsc

_LANES = 16
_PADW = 4
_TILE = 128
_T = 20


@functools.lru_cache(maxsize=None)
def _build(N: int, E: int, interpret: bool):
    try:
        info = plsc.get_sparse_core_info()
        NC, NS = info.num_cores, info.num_subcores
    except ValueError:
        NC, NS = 2, 16
    NW = NC * NS
    assert NW == 32
    assert E % (_TILE * _T) == 0, E
    NT = E // _TILE
    NCH = NT // _T
    C = _T * _TILE
    M = C // _LANES
    assert NCH // NW >= 4

    mesh = plsc.VectorSubcoreMesh(
        core_axis_name="c", subcore_axis_name="s", num_cores=NC, num_subcores=NS)

    vm = pltpu.VMEM
    dma = pltpu.SemaphoreType.DMA

    @functools.partial(
        pl.kernel,
        mesh=mesh,
        out_type=jax.ShapeDtypeStruct((NT, 4, _TILE), jnp.float32),
        scratch_types=(
            [vm((C,), jnp.int32)] * 4
            + [vm((C, _PADW), jnp.float32)] * 4
            + [vm((_T, 4, _TILE), jnp.float32)] * 2
            + [dma] * 8
        ),
        compiler_params=pltpu.CompilerParams(
            use_tc_tiling_on_sc=False, needs_layout_passes=False),
        interpret=interpret,
    )
    def k(pos_hbm, ii_hbm, ij_hbm, off_hbm, out_hbm,
          ii0, ij0, ii1, ij1, pi0, pj0, pi1, pj1, io0, io1,
          sx0, sx1, sg0, sg1, so0, so1, su0, su1):
        sets = ((ii0, ij0, pi0, pj0, io0, sx0, sg0, so0, su0),
                (ii1, ij1, pi1, pj1, io1, sx1, sg1, so1, su1))

        wid = lax.axis_index("s") * NC + lax.axis_index("c")
        c_lo = lax.shift_right_logical(wid * NCH, 5)
        c_hi = lax.shift_right_logical((wid + 1) * NCH, 5)
        n = c_hi - c_lo

        def issue_idx(c, S):
            pltpu.async_copy(ii_hbm.at[pl.ds(c * C, C)], S[0], S[5])
            pltpu.async_copy(ij_hbm.at[pl.ds(c * C, C)], S[1], S[5])

        def wait_idx(S):
            pltpu.make_async_copy(ii_hbm.at[pl.ds(0, C)], S[0], S[5]).wait()
            pltpu.make_async_copy(ij_hbm.at[pl.ds(0, C)], S[1], S[5]).wait()

        def issue_gath_off(c, S):
            pltpu.async_copy(pos_hbm.at[S[0]], S[2], S[6])
            pltpu.async_copy(pos_hbm.at[S[1]], S[3], S[6])
            pltpu.async_copy(off_hbm.at[pl.ds(c * _T, _T)], S[4], S[7])

        def wait_gath_off(S):
            pltpu.make_async_copy(pos_hbm.at[S[0]], S[2], S[6]).wait()
            pltpu.make_async_copy(pos_hbm.at[S[1]], S[3], S[6]).wait()
            pltpu.make_async_copy(off_hbm.at[pl.ds(0, _T)], S[4], S[7]).wait()

        def issue_out(c, S):
            pltpu.sync_copy(S[4], out_hbm.at[pl.ds(c * _T, _T)])

        def compute(S):
            pos_i_v, pos_j_v, io_v = S[2], S[3], S[4]

            def group_body(m, _):
                tt = lax.shift_right_logical(m, 3)
                lo = 16 * (m & 7)
                e_vec = m * _LANES + lax.iota(jnp.int32, _LANES)
                for s in range(3):
                    k_vec = jnp.broadcast_to(jnp.int32(s), (_LANES,))
                    pi = plsc.load_gather(pos_i_v, [e_vec, k_vec])
                    pj = plsc.load_gather(pos_j_v, [e_vec, k_vec])
                    io_v[tt, s, pl.ds(lo, _LANES)] = (
                        pj - pi + io_v[tt, s, pl.ds(lo, _LANES)])
                return 0

            lax.fori_loop(0, M, group_body, 0, unroll=False)

        def iteration(c, p, first):
            S, Q = sets[p], sets[1 - p]
            wait_gath_off(S)

            @pl.when(c + 2 < c_hi)
            def _():
                issue_idx(c + 2, S)

            compute(S)

            @pl.when(c + 1 < c_hi)
            def _():
                wait_idx(Q)
                issue_gath_off(c + 1, Q)

            issue_out(c, S)

        issue_idx(c_lo, sets[0])
        wait_idx(sets[0])
        issue_gath_off(c_lo, sets[0])
        issue_idx(c_lo + 1, sets[1])
        iteration(c_lo, 0, True)

        def pair_body(kk, _):
            i1 = 1 + 2 * kk
            iteration(c_lo + i1, 1, False)
            iteration(c_lo + i1 + 1, 0, False)
            return 0

        lax.fori_loop(0, lax.shift_right_logical(n - 1, 1), pair_body, 0,
                      unroll=False)

        @pl.when((n & 1) == 0)
        def _():
            iteration(c_hi - 1, 1, False)

    return k


def kernel(positions, indeces_i, indeces_j, offsets):
    N, _ = positions.shape
    E = indeces_i.shape[0]
    pos_pad = jnp.pad(positions, ((0, 0), (0, _PADW - positions.shape[1])))
    off_t = (jnp.pad(offsets, ((0, 0), (0, 1)))
             .reshape(E // _TILE, _TILE, 4)
             .transpose(0, 2, 1))
    k = _build(N, E, False)
    out_t = k(
        pos_pad,
        indeces_i.astype(jnp.int32),
        indeces_j.astype(jnp.int32),
        off_t,
    )
    return out_t.transpose(0, 2, 1).reshape(E, 4)[:, :3]


# speedup vs baseline: 1.0318x; 1.0282x over previous
import functools

import jax
import jax.numpy as jnp
from jax import lax
from jax.experimental import pallas as pl
from jax.experimental.pallas import tpu as pltpu
from jax.experimental.pallas import tpu_sc as plsc

_LANES = 16
_PADW = 4
_TILE = 128
_T = 20


@functools.lru_cache(maxsize=None)
def _build(N: int, E: int, interpret: bool):
    try:
        info = plsc.get_sparse_core_info()
        NC, NS = info.num_cores, info.num_subcores
    except ValueError:
        NC, NS = 2, 16
    NW = NC * NS
    assert NW == 32
    assert E % (_TILE * _T) == 0, E
    NT = E // _TILE
    NCH = NT // _T
    C = _T * _TILE
    M = C // _LANES
    assert NCH // NW >= 4

    mesh = plsc.VectorSubcoreMesh(
        core_axis_name="c", subcore_axis_name="s", num_cores=NC, num_subcores=NS)

    vm = pltpu.VMEM
    dma = pltpu.SemaphoreType.DMA

    @functools.partial(
        pl.kernel,
        mesh=mesh,
        out_type=jax.ShapeDtypeStruct((NT, 4, _TILE), jnp.float32),
        scratch_types=(
            [vm((C,), jnp.int32)] * 4
            + [vm((C, _PADW), jnp.float32)] * 4
            + [vm((_T, 4, _TILE), jnp.float32)] * 2
            + [dma] * 8
        ),
        compiler_params=pltpu.CompilerParams(
            use_tc_tiling_on_sc=False, needs_layout_passes=False),
        interpret=interpret,
    )
    def k(pos_hbm, ii_hbm, ij_hbm, off_hbm, out_hbm,
          ii0, ij0, ii1, ij1, pi0, pj0, pi1, pj1, io0, io1,
          sx0, sx1, sg0, sg1, so0, so1, su0, su1):
        sets = ((ii0, ij0, pi0, pj0, io0, sx0, sg0, so0, su0),
                (ii1, ij1, pi1, pj1, io1, sx1, sg1, so1, su1))

        wid = lax.axis_index("s") * NC + lax.axis_index("c")
        c_lo = lax.shift_right_logical(wid * NCH, 5)
        c_hi = lax.shift_right_logical((wid + 1) * NCH, 5)
        n = c_hi - c_lo

        def issue_idx(c, S):
            pltpu.async_copy(ii_hbm.at[pl.ds(c * C, C)], S[0], S[5])
            pltpu.async_copy(ij_hbm.at[pl.ds(c * C, C)], S[1], S[5])

        def wait_idx(S):
            pltpu.make_async_copy(ii_hbm.at[pl.ds(0, C)], S[0], S[5]).wait()
            pltpu.make_async_copy(ij_hbm.at[pl.ds(0, C)], S[1], S[5]).wait()

        def issue_gath(S):
            pltpu.async_copy(pos_hbm.at[S[0]], S[2], S[6])
            pltpu.async_copy(pos_hbm.at[S[1]], S[3], S[6])

        def issue_off(c, S):
            pltpu.async_copy(off_hbm.at[pl.ds(c * _T, _T)], S[4], S[7])

        def wait_gath_off(S):
            pltpu.make_async_copy(pos_hbm.at[S[0]], S[2], S[6]).wait()
            pltpu.make_async_copy(pos_hbm.at[S[1]], S[3], S[6]).wait()
            pltpu.make_async_copy(off_hbm.at[pl.ds(0, _T)], S[4], S[7]).wait()

        def issue_out(c, S):
            pltpu.async_copy(S[4], out_hbm.at[pl.ds(c * _T, _T)], S[8])

        def wait_out(S):
            pltpu.make_async_copy(
                S[4], out_hbm.at[pl.ds(0, _T)], S[8]).wait()

        def compute(S):
            pos_i_v, pos_j_v, io_v = S[2], S[3], S[4]

            def group_body(m, _):
                tt = lax.shift_right_logical(m, 3)
                lo = 16 * (m & 7)
                e_vec = m * _LANES + lax.iota(jnp.int32, _LANES)
                for s in range(3):
                    k_vec = jnp.broadcast_to(jnp.int32(s), (_LANES,))
                    pi = plsc.load_gather(pos_i_v, [e_vec, k_vec])
                    pj = plsc.load_gather(pos_j_v, [e_vec, k_vec])
                    io_v[tt, s, pl.ds(lo, _LANES)] = (
                        pj - pi + io_v[tt, s, pl.ds(lo, _LANES)])
                return 0

            lax.fori_loop(0, M, group_body, 0, unroll=2)

        def iteration(c, p, first):
            S, Q = sets[p], sets[1 - p]
            wait_gath_off(S)

            @pl.when(c + 1 < c_hi)
            def _():
                wait_idx(Q)
                if not first:
                    wait_out(Q)
                issue_off(c + 1, Q)

            @pl.when(c + 2 < c_hi)
            def _():
                issue_idx(c + 2, S)

            compute(S)

            @pl.when(c + 1 < c_hi)
            def _():
                issue_gath(Q)

            issue_out(c, S)

        issue_idx(c_lo, sets[0])
        wait_idx(sets[0])
        issue_gath(sets[0])
        issue_off(c_lo, sets[0])
        issue_idx(c_lo + 1, sets[1])
        iteration(c_lo, 0, True)

        def pair_body(kk, _):
            i1 = 1 + 2 * kk
            iteration(c_lo + i1, 1, False)
            iteration(c_lo + i1 + 1, 0, False)
            return 0

        lax.fori_loop(0, lax.shift_right_logical(n - 1, 1), pair_body, 0,
                      unroll=False)

        @pl.when((n & 1) == 0)
        def _():
            iteration(c_hi - 1, 1, False)

        wait_out(sets[0])
        wait_out(sets[1])

    return k


def kernel(positions, indeces_i, indeces_j, offsets):
    N, _ = positions.shape
    E = indeces_i.shape[0]
    pos_pad = jnp.pad(positions, ((0, 0), (0, _PADW - positions.shape[1])))
    off_t = (jnp.pad(offsets, ((0, 0), (0, 1)))
             .reshape(E // _TILE, _TILE, 4)
             .transpose(0, 2, 1))
    k = _build(N, E, False)
    out_t = k(
        pos_pad,
        indeces_i.astype(jnp.int32),
        indeces_j.astype(jnp.int32),
        off_t,
    )
    return out_t.transpose(0, 2, 1).reshape(E, 4)[:, :3]
